# baseline (speedup 1.0000x reference)
import functools

import jax
import jax.numpy as jnp
from jax.experimental import pallas as pl

B = 1024
NOBJ = 11
V = 4000
VSUB = 600
NBB = 8
NKP = 16
K = 208


def _w_builder_kernel(ang_ref, go_ref, tr_ref, qi_ref, w_ref):
    a = ang_ref[0:1, :]
    ca = jnp.cos(a * 0.5)
    sa = jnp.sin(a * 0.5)
    gx = go_ref[0:1, :]
    gy = go_ref[1:2, :]
    gz = go_ref[2:3, :]
    ang = jnp.sqrt(gx * gx + gy * gy + gz * gz)
    half = ang * 0.5
    small = jnp.abs(ang) < 1e-6
    safe = jnp.where(small, jnp.ones_like(ang), ang)
    sho = jnp.where(small, 0.5 - ang * ang / 48.0, jnp.sin(half) / safe)
    qw = jnp.cos(half)
    qx = gx * sho
    qy = gy * sho
    qz = gz * sho
    tw = qw * ca + qz * sa
    tx = qx * ca - qy * sa
    ty = qy * ca + qx * sa
    tz = qz * ca - qw * sa

    def mat_rows(w, x, y, z):
        m = [[1 - 2 * (y * y + z * z), 2 * (x * y - w * z), 2 * (x * z + w * y)],
             [2 * (x * y + w * z), 1 - 2 * (x * x + z * z), 2 * (y * z - w * x)],
             [2 * (x * z - w * y), 2 * (y * z + w * x), 1 - 2 * (x * x + y * y)]]
        return jnp.concatenate([m[j][k] for k in range(3) for j in range(3)], axis=0)

    top9 = mat_rows(tw, tx, ty, tz)
    bot9 = mat_rows(qw, qx, qy, qz)
    qi = qi_ref[0:1, :]
    for o in range(NOBJ):
        mask = (qi == o).astype(jnp.float32)
        w_ref[o * 9:(o + 1) * 9, :] = top9 * mask
        w_ref[102 + o * 9:102 + (o + 1) * 9, :] = bot9 * mask
    w_ref[99:102, :] = tr_ref[...]
    w_ref[201:K, :] = jnp.zeros((K - 201, B), jnp.float32)


def _matmul_kernel(w_ref, r2v_ref, r2vs_ref, r2sm_ref, v_ref, vs_ref, sm_ref):
    w = w_ref[...]
    v_ref[...] = jnp.dot(w, r2v_ref[...], preferred_element_type=jnp.float32)
    vs_ref[...] = jnp.dot(w, r2vs_ref[...], preferred_element_type=jnp.float32)
    sm_ref[...] = jnp.dot(w, r2sm_ref[...], preferred_element_type=jnp.float32)


def _build_r2_masked(tab, mask_top):
    n = tab.shape[1]
    eye = jnp.eye(3, dtype=jnp.float32)
    core = jnp.einsum('ovk,jp->okjvp', tab, eye)
    mt = mask_top.astype(jnp.float32)[:, None, None, :, None]
    r2_top = (core * mt).reshape(NOBJ * 9, n * 3)
    r2_bot = (core * (1.0 - mt)).reshape(NOBJ * 9, n * 3)
    trows = jnp.tile(eye, (1, n))
    return jnp.concatenate(
        [r2_top, trows, r2_bot, jnp.zeros((K - 201, n * 3), jnp.float32)], axis=0)


def _build_r2_pair(tab_top, tab_bot):
    n = tab_top.shape[1]
    eye = jnp.eye(3, dtype=jnp.float32)
    core_t = jnp.einsum('ovk,jp->okjvp', tab_top, eye).reshape(NOBJ * 9, n * 3)
    core_b = jnp.einsum('ovk,jp->okjvp', tab_bot, eye).reshape(NOBJ * 9, n * 3)
    z = jnp.zeros((NOBJ * 9, n * 3), jnp.float32)
    top_rows = jnp.concatenate([core_t, z], axis=1)
    bot_rows = jnp.concatenate([z, core_b], axis=1)
    trows = jnp.tile(eye, (1, 2 * n))
    return jnp.concatenate(
        [top_rows, trows, bot_rows, jnp.zeros((K - 201, 2 * n * 3), jnp.float32)],
        axis=0)


@functools.partial(jax.jit, static_argnames=())
def kernel(angles, global_orient, transl, query_idx, v, v_sub, bbox_top,
           bbox_bottom, kp_top, kp_bottom, parts_ids, parts_sub_ids):
    ang_t = angles.reshape(B, 1).T
    go_t = global_orient.T
    tr_t = transl.T
    qi_t = query_idx.astype(jnp.int32).reshape(B, 1).T

    w_t = pl.pallas_call(
        _w_builder_kernel,
        out_shape=jax.ShapeDtypeStruct((K, B), jnp.float32),
    )(ang_t, go_t, tr_t, qi_t)
    w = w_t.T

    r2v = _build_r2_masked(v, parts_ids == 1)
    r2vs = _build_r2_masked(v_sub, parts_sub_ids == 1)
    r2bb = _build_r2_pair(bbox_top, bbox_bottom)
    r2kp = _build_r2_pair(kp_top, kp_bottom)
    r2sm = jnp.concatenate([r2bb, r2kp], axis=1)

    mv, mvs, msm = 3 * V, 3 * VSUB, 3 * (2 * NBB + 2 * NKP)
    bb = 128
    grid = (B // bb,)
    v_flat, vs_flat, sm_flat = pl.pallas_call(
        _matmul_kernel,
        grid=grid,
        in_specs=[
            pl.BlockSpec((bb, K), lambda i: (i, 0)),
            pl.BlockSpec((K, mv), lambda i: (0, 0)),
            pl.BlockSpec((K, mvs), lambda i: (0, 0)),
            pl.BlockSpec((K, msm), lambda i: (0, 0)),
        ],
        out_specs=[
            pl.BlockSpec((bb, mv), lambda i: (i, 0)),
            pl.BlockSpec((bb, mvs), lambda i: (i, 0)),
            pl.BlockSpec((bb, msm), lambda i: (i, 0)),
        ],
        out_shape=[
            jax.ShapeDtypeStruct((B, mv), jnp.float32),
            jax.ShapeDtypeStruct((B, mvs), jnp.float32),
            jax.ShapeDtypeStruct((B, msm), jnp.float32),
        ],
    )(w, r2v, r2vs, r2sm)

    v_out = v_flat.reshape(B, V, 3)
    vs_out = vs_flat.reshape(B, VSUB, 3)
    bbox3d = sm_flat[:, :3 * 2 * NBB].reshape(B, 2 * NBB, 3)
    kp3d = sm_flat[:, 3 * 2 * NBB:].reshape(B, 2 * NKP, 3)
    return v_out, vs_out, bbox3d, kp3d


# speedup vs baseline: 1.7312x; 1.7312x over previous
import functools

import jax
import jax.numpy as jnp
from jax.experimental import pallas as pl

B = 1024
NOBJ = 11
V = 4000
VSUB = 600
NBB = 8
NKP = 16
K = 208


def _w_builder_kernel(ang_ref, go_ref, tr_ref, qi_ref, w_ref):
    a = ang_ref[0:1, :]
    ca = jnp.cos(a * 0.5)
    sa = jnp.sin(a * 0.5)
    gx = go_ref[0:1, :]
    gy = go_ref[1:2, :]
    gz = go_ref[2:3, :]
    ang = jnp.sqrt(gx * gx + gy * gy + gz * gz)
    half = ang * 0.5
    small = jnp.abs(ang) < 1e-6
    safe = jnp.where(small, jnp.ones_like(ang), ang)
    sho = jnp.where(small, 0.5 - ang * ang / 48.0, jnp.sin(half) / safe)
    qw = jnp.cos(half)
    qx = gx * sho
    qy = gy * sho
    qz = gz * sho
    tw = qw * ca + qz * sa
    tx = qx * ca - qy * sa
    ty = qy * ca + qx * sa
    tz = qz * ca - qw * sa

    def mat(w, x, y, z):
        return [[1 - 2 * (y * y + z * z), 2 * (x * y - w * z), 2 * (x * z + w * y)],
                [2 * (x * y + w * z), 1 - 2 * (x * x + z * z), 2 * (y * z - w * x)],
                [2 * (x * z - w * y), 2 * (y * z + w * x), 1 - 2 * (x * x + y * y)]]

    mt = mat(tw, tx, ty, tz)
    mb = mat(qw, qx, qy, qz)
    qi = qi_ref[0:1, :]
    oids = jax.lax.broadcasted_iota(jnp.int32, (NOBJ, B), 0)
    onehot = (oids == qi).astype(jnp.float32)
    for k in range(3):
        for j in range(3):
            c = (k * 3 + j) * NOBJ
            w_ref[c:c + NOBJ, :] = onehot * mt[j][k]
            w_ref[102 + c:102 + c + NOBJ, :] = onehot * mb[j][k]
    w_ref[99:102, :] = tr_ref[...]
    w_ref[201:K, :] = jnp.zeros((K - 201, B), jnp.float32)


def _matmul_kernel(w_ref, r2v_ref, r2vs_ref, r2sm_ref, v_ref, vs_ref, sm_ref):
    w = w_ref[...]
    v_ref[...] = jnp.dot(w, r2v_ref[...], preferred_element_type=jnp.float32)
    vs_ref[...] = jnp.dot(w, r2vs_ref[...], preferred_element_type=jnp.float32)
    sm_ref[...] = jnp.dot(w, r2sm_ref[...], preferred_element_type=jnp.float32)


def _roll_blocks(flat_top, flat_bot, m):
    lane = jax.lax.broadcasted_iota(jnp.int32, (1, m), 1) % 3
    tops, bots = [], []
    for k in range(3):
        for j in range(3):
            lm = (lane == j).astype(jnp.float32)
            tops.append(jnp.roll(flat_top, j - k, axis=1) * lm)
            bots.append(jnp.roll(flat_bot, j - k, axis=1) * lm)
    trows = jnp.concatenate(
        [(lane == jj).astype(jnp.float32) for jj in range(3)], axis=0)
    return jnp.concatenate(tops, axis=0), trows, jnp.concatenate(bots, axis=0)


def _build_r2_masked(tab, mask_top, n):
    m = 3 * n
    mt = mask_top.astype(jnp.float32)[..., None]
    flat_top = (tab * mt).reshape(NOBJ, m)
    flat_bot = (tab * (1.0 - mt)).reshape(NOBJ, m)
    top99, trows, bot99 = _roll_blocks(flat_top, flat_bot, m)
    return jnp.concatenate(
        [top99, trows, bot99, jnp.zeros((K - 201, m), jnp.float32)], axis=0)


def _build_r2_pair(tab_top, tab_bot, n):
    m = 3 * n
    z = jnp.zeros((NOBJ, m), jnp.float32)
    ft = jnp.concatenate([tab_top.reshape(NOBJ, m), z], axis=1)
    fb = jnp.concatenate([z, tab_bot.reshape(NOBJ, m)], axis=1)
    top99, trows, bot99 = _roll_blocks(ft, fb, 2 * m)
    return jnp.concatenate(
        [top99, trows, bot99, jnp.zeros((K - 201, 2 * m), jnp.float32)], axis=0)


@functools.partial(jax.jit, static_argnames=())
def kernel(angles, global_orient, transl, query_idx, v, v_sub, bbox_top,
           bbox_bottom, kp_top, kp_bottom, parts_ids, parts_sub_ids):
    ang_t = angles.reshape(B, 1).T
    go_t = global_orient.T
    tr_t = transl.T
    qi_t = query_idx.astype(jnp.int32).reshape(B, 1).T

    w_t = pl.pallas_call(
        _w_builder_kernel,
        out_shape=jax.ShapeDtypeStruct((K, B), jnp.float32),
    )(ang_t, go_t, tr_t, qi_t)
    w = w_t.T

    r2v = _build_r2_masked(v, parts_ids == 1, V)
    r2vs = _build_r2_masked(v_sub, parts_sub_ids == 1, VSUB)
    r2bb = _build_r2_pair(bbox_top, bbox_bottom, NBB)
    r2kp = _build_r2_pair(kp_top, kp_bottom, NKP)
    r2sm = jnp.concatenate([r2bb, r2kp], axis=1)

    mv, mvs, msm = 3 * V, 3 * VSUB, 3 * (2 * NBB + 2 * NKP)
    bb = 128
    grid = (B // bb,)
    v_flat, vs_flat, sm_flat = pl.pallas_call(
        _matmul_kernel,
        grid=grid,
        in_specs=[
            pl.BlockSpec((bb, K), lambda i: (i, 0)),
            pl.BlockSpec((K, mv), lambda i: (0, 0)),
            pl.BlockSpec((K, mvs), lambda i: (0, 0)),
            pl.BlockSpec((K, msm), lambda i: (0, 0)),
        ],
        out_specs=[
            pl.BlockSpec((bb, mv), lambda i: (i, 0)),
            pl.BlockSpec((bb, mvs), lambda i: (i, 0)),
            pl.BlockSpec((bb, msm), lambda i: (i, 0)),
        ],
        out_shape=[
            jax.ShapeDtypeStruct((B, mv), jnp.float32),
            jax.ShapeDtypeStruct((B, mvs), jnp.float32),
            jax.ShapeDtypeStruct((B, msm), jnp.float32),
        ],
    )(w, r2v, r2vs, r2sm)

    v_out = v_flat.reshape(B, V, 3)
    vs_out = vs_flat.reshape(B, VSUB, 3)
    bbox3d = sm_flat[:, :3 * 2 * NBB].reshape(B, 2 * NBB, 3)
    kp3d = sm_flat[:, 3 * 2 * NBB:].reshape(B, 2 * NKP, 3)
    return v_out, vs_out, bbox3d, kp3d


# speedup vs baseline: 29.6868x; 17.1479x over previous
import functools

import jax
import jax.numpy as jnp
from jax.experimental import pallas as pl

B = 1024
NOBJ = 11
V = 4000
VSUB = 600
NBB = 8
NKP = 16
KC = 72


def _w3_builder_kernel(ang_ref, go_ref, tr_ref, qi_ref, w_ref):
    a = ang_ref[0:1, :]
    ca = jnp.cos(a * 0.5)
    sa = jnp.sin(a * 0.5)
    gx = go_ref[0:1, :]
    gy = go_ref[1:2, :]
    gz = go_ref[2:3, :]
    ang = jnp.sqrt(gx * gx + gy * gy + gz * gz)
    half = ang * 0.5
    small = jnp.abs(ang) < 1e-6
    safe = jnp.where(small, jnp.ones_like(ang), ang)
    sho = jnp.where(small, 0.5 - ang * ang / 48.0, jnp.sin(half) / safe)
    qw = jnp.cos(half)
    qx = gx * sho
    qy = gy * sho
    qz = gz * sho
    tw = qw * ca + qz * sa
    tx = qx * ca - qy * sa
    ty = qy * ca + qx * sa
    tz = qz * ca - qw * sa

    def mat(w, x, y, z):
        return [[1 - 2 * (y * y + z * z), 2 * (x * y - w * z), 2 * (x * z + w * y)],
                [2 * (x * y + w * z), 1 - 2 * (x * x + z * z), 2 * (y * z - w * x)],
                [2 * (x * z - w * y), 2 * (y * z + w * x), 1 - 2 * (x * x + y * y)]]

    mt = mat(tw, tx, ty, tz)
    mb = mat(qw, qx, qy, qz)
    qi = qi_ref[0:1, :]
    oids = jax.lax.broadcasted_iota(jnp.int32, (NOBJ, B), 0)
    onehot = (oids == qi).astype(jnp.float32)
    zrow = jnp.zeros((KC - 67, B), jnp.float32)
    for p in range(3):
        r = p * KC
        for k in range(3):
            w_ref[r + k * NOBJ:r + (k + 1) * NOBJ, :] = onehot * mt[p][k]
            w_ref[r + 34 + k * NOBJ:r + 34 + (k + 1) * NOBJ, :] = onehot * mb[p][k]
        w_ref[r + 33:r + 34, :] = tr_ref[p:p + 1, :]
        w_ref[r + 67:r + KC, :] = zrow


def _mm_kernel(w_ref, xv_ref, xvs_ref, xbb_ref, xkp_ref,
               ov_ref, ovs_ref, obb_ref, okp_ref):
    w = w_ref[...]
    f32 = jnp.float32
    ov_ref[...] = jnp.dot(xv_ref[...], w, preferred_element_type=f32)
    ovs_ref[...] = jnp.dot(xvs_ref[...], w, preferred_element_type=f32)
    obb_ref[...] = jnp.dot(xbb_ref[...], w, preferred_element_type=f32)
    okp_ref[...] = jnp.dot(xkp_ref[...], w, preferred_element_type=f32)


def _x_masked(tab, mask_top, n):
    mt = mask_top.astype(jnp.float32)[..., None]
    t33 = (tab * mt).transpose(1, 2, 0).reshape(n, 3 * NOBJ)
    b33 = (tab * (1.0 - mt)).transpose(1, 2, 0).reshape(n, 3 * NOBJ)
    ones = jnp.ones((n, 1), jnp.float32)
    pad = jnp.zeros((n, KC - 67), jnp.float32)
    return jnp.concatenate([t33, ones, b33, pad], axis=1)


def _x_pair(tab_top, tab_bot, n):
    t33 = tab_top.transpose(1, 2, 0).reshape(n, 3 * NOBJ)
    b33 = tab_bot.transpose(1, 2, 0).reshape(n, 3 * NOBJ)
    z = jnp.zeros((n, 3 * NOBJ), jnp.float32)
    ones = jnp.ones((n, 1), jnp.float32)
    pad = jnp.zeros((n, KC - 67), jnp.float32)
    top_rows = jnp.concatenate([t33, ones, z, pad], axis=1)
    bot_rows = jnp.concatenate([z, ones, b33, pad], axis=1)
    return jnp.concatenate([top_rows, bot_rows], axis=0)


@functools.partial(jax.jit, static_argnames=())
def kernel(angles, global_orient, transl, query_idx, v, v_sub, bbox_top,
           bbox_bottom, kp_top, kp_bottom, parts_ids, parts_sub_ids):
    ang_t = angles.reshape(B, 1).T
    go_t = global_orient.T
    tr_t = transl.T
    qi_t = query_idx.astype(jnp.int32).reshape(B, 1).T

    w3 = pl.pallas_call(
        _w3_builder_kernel,
        out_shape=jax.ShapeDtypeStruct((3 * KC, B), jnp.float32),
    )(ang_t, go_t, tr_t, qi_t)

    xv = _x_masked(v, parts_ids == 1, V)
    xvs = _x_masked(v_sub, parts_sub_ids == 1, VSUB)
    xbb = _x_pair(bbox_top, bbox_bottom, NBB)
    xkp = _x_pair(kp_top, kp_bottom, NKP)

    bn = 512
    grid = (3, B // bn)
    ov, ovs, obb, okp = pl.pallas_call(
        _mm_kernel,
        grid=grid,
        in_specs=[
            pl.BlockSpec((KC, bn), lambda p, b: (p, b)),
            pl.BlockSpec((V, KC), lambda p, b: (0, 0)),
            pl.BlockSpec((VSUB, KC), lambda p, b: (0, 0)),
            pl.BlockSpec((2 * NBB, KC), lambda p, b: (0, 0)),
            pl.BlockSpec((2 * NKP, KC), lambda p, b: (0, 0)),
        ],
        out_specs=[
            pl.BlockSpec((V, bn), lambda p, b: (p, b)),
            pl.BlockSpec((VSUB, bn), lambda p, b: (p, b)),
            pl.BlockSpec((2 * NBB, bn), lambda p, b: (p, b)),
            pl.BlockSpec((2 * NKP, bn), lambda p, b: (p, b)),
        ],
        out_shape=[
            jax.ShapeDtypeStruct((3 * V, B), jnp.float32),
            jax.ShapeDtypeStruct((3 * VSUB, B), jnp.float32),
            jax.ShapeDtypeStruct((3 * 2 * NBB, B), jnp.float32),
            jax.ShapeDtypeStruct((3 * 2 * NKP, B), jnp.float32),
        ],
    )(w3, xv, xvs, xbb, xkp)

    v_out = jnp.transpose(ov.reshape(3, V, B), (2, 1, 0))
    vs_out = jnp.transpose(ovs.reshape(3, VSUB, B), (2, 1, 0))
    bbox3d = jnp.transpose(obb.reshape(3, 2 * NBB, B), (2, 1, 0))
    kp3d = jnp.transpose(okp.reshape(3, 2 * NKP, B), (2, 1, 0))
    return v_out, vs_out, bbox3d, kp3d
